# baseline (speedup 1.0000x reference)
import dataclasses
import functools

import jax
import jax.numpy as jnp
from jax import lax
from jax.experimental import pallas as pl
from jax.experimental.pallas import tpu as pltpu
from jax.experimental.pallas import tpu_sc as plsc

N = 10000
E = 320000
D = 128
H = 8
DH = D // H
ACC_W = D + DH

KV_W = D // 2 + D

NC = 2
NS = 16
NW = NC * NS
C = 40
EPT = E // NW
CPT = EPT // C
ROWS_PER_TILE = N // NS
ZERO_STEP = 25
OUT_STEP = 125


def _project_kv(kv, W_k, b_k, W_v, b_v):
    BR = 1000

    def body(x_ref, wk_ref, bk_ref, wv_ref, bv_ref, o_ref):
        x = x_ref[...]
        k = lax.dot_general(x, wk_ref[...], (((1,), (1,)), ((), ())),
                            preferred_element_type=jnp.float32)
        v = lax.dot_general(x, wv_ref[...], (((1,), (1,)), ((), ())),
                            preferred_element_type=jnp.float32)
        o_ref[...] = jnp.concatenate(
            [(k + bk_ref[...]) * (1.0 / (DH ** 0.5)), v + bv_ref[...]], axis=1)

    return pl.pallas_call(
        body,
        grid=(N // BR,),
        in_specs=[pl.BlockSpec((BR, D), lambda i: (i, 0)),
                  pl.BlockSpec((D, D), lambda i: (0, 0)),
                  pl.BlockSpec((1, D), lambda i: (0, 0)),
                  pl.BlockSpec((D, D), lambda i: (0, 0)),
                  pl.BlockSpec((1, D), lambda i: (0, 0))],
        out_specs=pl.BlockSpec((BR, 2 * D), lambda i: (i, 0)),
        out_shape=jax.ShapeDtypeStruct((N, 2 * D), jnp.float32),
    )(kv, W_k, b_k.reshape(1, D), W_v, b_v.reshape(1, D))


def _sc_attention(q_tab, kv_tab, edge_index):
    mesh = plsc.VectorSubcoreMesh(core_axis_name="c", subcore_axis_name="s")
    cp = pltpu.CompilerParams()
    if "needs_layout_passes" in pltpu.CompilerParams.__dataclass_fields__:
        cp = dataclasses.replace(cp, needs_layout_passes=False)
    if "use_tc_tiling_on_sc" in pltpu.CompilerParams.__dataclass_fields__:
        cp = dataclasses.replace(cp, use_tc_tiling_on_sc=False)

    @functools.partial(
        pl.kernel,
        out_type=jax.ShapeDtypeStruct((NC, N, ACC_W), jnp.float32),
        mesh=mesh,
        compiler_params=cp,
        scratch_types=[
            pltpu.VMEM((2, C), jnp.int32),
            pltpu.VMEM((2, C), jnp.int32),
            pltpu.VMEM((C, KV_W), jnp.float32),
            pltpu.VMEM((C, KV_W), jnp.float32),
            pltpu.VMEM((C, D // 2), jnp.float32),
            pltpu.VMEM((C, D // 2), jnp.float32),
            pltpu.VMEM((C, ACC_W), jnp.float32),
            pltpu.VMEM((1, C), jnp.int32),
            pltpu.VMEM((1, C), jnp.int32),
            pltpu.VMEM_SHARED((N, ACC_W), jnp.float32),
            pltpu.SemaphoreType.DMA,
            pltpu.SemaphoreType.DMA,
            pltpu.SemaphoreType.DMA,
            pltpu.SemaphoreType.DMA,
            pltpu.SemaphoreType.DMA,
            pltpu.SemaphoreType.DMA,
            pltpu.SemaphoreType.DMA,
            pltpu.SemaphoreType.DMA,
            pltpu.SemaphoreType.DMA,
        ],
    )
    def sc_kernel(q_hbm, kv_hbm, ei_hbm, out_hbm, idx0, idx1, kvb0, kvb1,
                  qb0, qb1, mb, sd0, sd1, acc, semI0, semI1, semG0, semG1,
                  semQ0, semQ1, semD0, semD1, semS):
        cid = lax.axis_index("c")
        sid = lax.axis_index("s")
        wid = cid * NS + sid
        lane = lax.iota(jnp.int32, 16)
        dnums = lax.GatherDimensionNumbers(
            offset_dims=(), collapsed_slice_dims=(0,), start_index_map=(0,))
        perms = [jnp.bitwise_xor(lane, s)[:, None] for s in (8, 4, 2, 1)]

        def _dot_splat(x):
            for p in perms:
                x = x + lax.gather(x, p, dnums, (1,),
                                   mode=lax.GatherScatterMode.PROMISE_IN_BOUNDS)
            return x

        @pl.loop(0, ZERO_STEP)
        def _(r):
            @pl.loop(0, ACC_W, step=DH)
            def _(cc):
                mb[r, pl.ds(cc, DH)] = jnp.zeros((DH,), jnp.float32)

        row0 = sid * ROWS_PER_TILE

        @pl.loop(0, ROWS_PER_TILE, step=ZERO_STEP)
        def _(r):
            pltpu.sync_copy(mb.at[pl.ds(0, ZERO_STEP)],
                            acc.at[pl.ds(row0 + r, ZERO_STEP)])

        plsc.subcore_barrier()

        ebase = wid * EPT
        bufs = ((idx0, kvb0, qb0, sd0, semI0, semG0, semQ0, semD0),
                (idx1, kvb1, qb1, sd1, semI1, semG1, semQ1, semD1))

        pltpu.async_copy(ei_hbm.at[pl.ds(0, 2), pl.ds(ebase, C)],
                         idx0, semI0).wait()
        pltpu.async_copy(ei_hbm.at[pl.ds(0, 2), pl.ds(ebase + C, C)],
                         idx1, semI1)
        pltpu.async_copy(kv_hbm.at[idx0.at[0]], kvb0, semG0)
        pltpu.async_copy(q_hbm.at[idx0.at[1]], qb0, semQ0)

        @pl.loop(0, CPT, step=2)
        def _(i):
            for b in range(2):
                ii = i + b
                idxb, kvbb, qbb, sdb, semIb, semGb, semQb, semDb = bufs[b]
                idxn, kvbn, qbn, sdn, semIn, semGn, semQn, semDn = bufs[1 - b]

                pltpu.async_copy(
                    ei_hbm.at[pl.ds(1, 1), pl.ds(ebase + ii * C, C)],
                    sdb, semDb)

                pltpu.make_async_copy(kv_hbm.at[idxb.at[0]], kvbb, semGb).wait()
                pltpu.make_async_copy(q_hbm.at[idxb.at[1]], qbb, semQb).wait()

                @pl.when(ii + 2 < CPT)
                def _():
                    pltpu.async_copy(
                        ei_hbm.at[pl.ds(0, 2), pl.ds(ebase + (ii + 2) * C, C)],
                        idxb, semIb)

                @pl.when(ii + 1 < CPT)
                def _():
                    pltpu.make_async_copy(
                        ei_hbm.at[pl.ds(0, 2), pl.ds(ebase, C)],
                        idxn, semIn).wait()
                    pltpu.async_copy(kv_hbm.at[idxn.at[0]], kvbn, semGn)
                    pltpu.async_copy(q_hbm.at[idxn.at[1]], qbn, semQn)

                @pl.when(ii > 0)
                def _():
                    pltpu.make_async_copy(mb, acc.at[sdn.at[0]], semS).wait()

                @plsc.parallel_loop(0, C, unroll=8)
                def _(e):
                    den = jnp.zeros((DH, ), jnp.float32)
                    for p in range(H // 2):
                        kpair = plsc.bitcast(kvbb[e, pl.ds(DH * p, DH)],
                                             jnp.bfloat16)
                        qpair = plsc.bitcast(qbb[e, pl.ds(DH * p, DH)],
                                             jnp.bfloat16)
                        kk = plsc.unpack(kpair, format=plsc.PackFormat.INTERLEAVED)
                        qq = plsc.unpack(qpair, format=plsc.PackFormat.INTERLEAVED)
                        for s in range(2):
                            h = 2 * p + s
                            evec = jnp.exp(jnp.broadcast_to(
                                jnp.sum(kk[s] * qq[s]), (DH,)))
                            vvec = kvbb[e, pl.ds(D // 2 + h * DH, DH)]
                            mb[e, pl.ds(h * DH, DH)] = vvec * evec
                            den = jnp.where(lane == h, evec, den)
                    mb[e, pl.ds(D, DH)] = den

                pltpu.make_async_copy(
                    ei_hbm.at[pl.ds(1, 1), pl.ds(ebase, C)], sdb, semDb).wait()
                pltpu.async_copy(mb, acc.at[sdb.at[0]], semS, add=True)

        pltpu.make_async_copy(mb, acc.at[sd1.at[0]], semS).wait()
        plsc.subcore_barrier()

        @pl.loop(0, ROWS_PER_TILE, step=OUT_STEP)
        def _(r):
            pltpu.sync_copy(acc.at[pl.ds(row0 + r, OUT_STEP)],
                            out_hbm.at[cid, pl.ds(row0 + r, OUT_STEP)])

    return sc_kernel(q_tab, kv_tab, edge_index)


def _combine(parts, W_o, b_o):
    BR = 1000

    def body(p_ref, wo_ref, bo_ref, o_ref):
        full = p_ref[0] + p_ref[1]
        msg = full[:, :D]
        r_ = lax.broadcasted_iota(jnp.int32, (ACC_W, D), 0)
        c_ = lax.broadcasted_iota(jnp.int32, (ACC_W, D), 1)
        expand = jnp.where((r_ >= D) & (c_ // DH == r_ - D), 1.0, 0.0)
        dexp = lax.dot_general(full, expand, (((1,), (0,)), ((), ())),
                               preferred_element_type=jnp.float32)
        attn_msg = msg / (dexp + 1e-16)
        out = lax.dot_general(attn_msg, wo_ref[...], (((1,), (1,)), ((), ())),
                              preferred_element_type=jnp.float32)
        o_ref[...] = out + bo_ref[...]

    return pl.pallas_call(
        body,
        grid=(N // BR,),
        in_specs=[pl.BlockSpec((NC, BR, ACC_W), lambda i: (0, i, 0)),
                  pl.BlockSpec((D, D), lambda i: (0, 0)),
                  pl.BlockSpec((1, D), lambda i: (0, 0))],
        out_specs=pl.BlockSpec((BR, D), lambda i: (i, 0)),
        out_shape=jax.ShapeDtypeStruct((N, D), jnp.float32),
    )(parts, W_o, b_o.reshape(1, D))


def _pack_pairs(x):
    n = x.shape[0]
    pairs = x.reshape(n, 4, 2, DH).transpose(0, 1, 3, 2).reshape(n, 64, 2)
    return jax.lax.bitcast_convert_type(pairs.astype(jnp.bfloat16), jnp.float32)


def kernel(q, kv, edge_index, W_k, b_k, W_v, b_v, W_o, b_o):
    kv_tab = _project_kv(kv, W_k, b_k, W_v, b_v)
    kv_mix = jnp.concatenate([_pack_pairs(kv_tab[:, :D]), kv_tab[:, D:]], axis=1)
    parts = _sc_attention(_pack_pairs(q), kv_mix, edge_index)
    return _combine(parts, W_o, b_o)


# speedup vs baseline: 2.5975x; 2.5975x over previous
import dataclasses
import functools

import jax
import jax.numpy as jnp
from jax import lax
from jax.experimental import pallas as pl
from jax.experimental.pallas import tpu as pltpu
from jax.experimental.pallas import tpu_sc as plsc

N = 10000
E = 320000
D = 128
H = 8
DH = D // H
ACC_W = D + DH

KV_W = D // 2 + D

NC = 2
NS = 16
NW = NC * NS
C = 40
EPT = E // NW
CPT = EPT // C
ROWS_PER_TILE = N // NS
ZERO_STEP = 25
OUT_STEP = 125


def _project_kv(kv, W_k, b_k, W_v, b_v):
    BR = 1000

    def body(x_ref, wk_ref, bk_ref, wv_ref, bv_ref, o_ref):
        x = x_ref[...]
        k = lax.dot_general(x, wk_ref[...], (((1,), (1,)), ((), ())),
                            preferred_element_type=jnp.float32)
        v = lax.dot_general(x, wv_ref[...], (((1,), (1,)), ((), ())),
                            preferred_element_type=jnp.float32)
        o_ref[...] = jnp.concatenate(
            [(k + bk_ref[...]) * (1.0 / (DH ** 0.5)), v + bv_ref[...]], axis=1)

    return pl.pallas_call(
        body,
        grid=(N // BR,),
        in_specs=[pl.BlockSpec((BR, D), lambda i: (i, 0)),
                  pl.BlockSpec((D, D), lambda i: (0, 0)),
                  pl.BlockSpec((1, D), lambda i: (0, 0)),
                  pl.BlockSpec((D, D), lambda i: (0, 0)),
                  pl.BlockSpec((1, D), lambda i: (0, 0))],
        out_specs=pl.BlockSpec((BR, 2 * D), lambda i: (i, 0)),
        out_shape=jax.ShapeDtypeStruct((N, 2 * D), jnp.float32),
    )(kv, W_k, b_k.reshape(1, D), W_v, b_v.reshape(1, D))


def _sc_attention(q_tab, kv_tab, edge_index):
    mesh = plsc.VectorSubcoreMesh(core_axis_name="c", subcore_axis_name="s")
    cp = pltpu.CompilerParams()
    if "needs_layout_passes" in pltpu.CompilerParams.__dataclass_fields__:
        cp = dataclasses.replace(cp, needs_layout_passes=False)
    if "use_tc_tiling_on_sc" in pltpu.CompilerParams.__dataclass_fields__:
        cp = dataclasses.replace(cp, use_tc_tiling_on_sc=False)

    @functools.partial(
        pl.kernel,
        out_type=jax.ShapeDtypeStruct((NC, N, ACC_W), jnp.float32),
        mesh=mesh,
        compiler_params=cp,
        scratch_types=[
            pltpu.VMEM((2, C), jnp.int32),
            pltpu.VMEM((2, C), jnp.int32),
            pltpu.VMEM((C, KV_W), jnp.float32),
            pltpu.VMEM((C, KV_W), jnp.float32),
            pltpu.VMEM((C, D // 2), jnp.float32),
            pltpu.VMEM((C, D // 2), jnp.float32),
            pltpu.VMEM((C, ACC_W), jnp.float32),
            pltpu.VMEM((1, C), jnp.int32),
            pltpu.VMEM((1, C), jnp.int32),
            pltpu.VMEM_SHARED((N, ACC_W), jnp.float32),
            pltpu.SemaphoreType.DMA,
            pltpu.SemaphoreType.DMA,
            pltpu.SemaphoreType.DMA,
            pltpu.SemaphoreType.DMA,
            pltpu.SemaphoreType.DMA,
            pltpu.SemaphoreType.DMA,
            pltpu.SemaphoreType.DMA,
            pltpu.SemaphoreType.DMA,
            pltpu.SemaphoreType.DMA,
        ],
    )
    def sc_kernel(q_hbm, kv_hbm, ei_hbm, out_hbm, idx0, idx1, kvb0, kvb1,
                  qb0, qb1, mb, sd0, sd1, acc, semI0, semI1, semG0, semG1,
                  semQ0, semQ1, semD0, semD1, semS):
        cid = lax.axis_index("c")
        sid = lax.axis_index("s")
        wid = cid * NS + sid
        lane = lax.iota(jnp.int32, 16)
        dnums = lax.GatherDimensionNumbers(
            offset_dims=(), collapsed_slice_dims=(0,), start_index_map=(0,))
        perms = [jnp.bitwise_xor(lane, s)[:, None] for s in (8, 4, 2, 1)]

        def _dot_splat(x):
            for p in perms:
                x = x + lax.gather(x, p, dnums, (1,),
                                   mode=lax.GatherScatterMode.PROMISE_IN_BOUNDS)
            return x

        @pl.loop(0, ZERO_STEP)
        def _(r):
            @pl.loop(0, ACC_W, step=DH)
            def _(cc):
                mb[r, pl.ds(cc, DH)] = jnp.zeros((DH,), jnp.float32)

        row0 = sid * ROWS_PER_TILE

        @pl.loop(0, ROWS_PER_TILE, step=ZERO_STEP)
        def _(r):
            pltpu.sync_copy(mb.at[pl.ds(0, ZERO_STEP)],
                            acc.at[pl.ds(row0 + r, ZERO_STEP)])

        plsc.subcore_barrier()

        ebase = wid * EPT
        bufs = ((idx0, kvb0, qb0, sd0, semI0, semG0, semQ0, semD0),
                (idx1, kvb1, qb1, sd1, semI1, semG1, semQ1, semD1))

        pltpu.async_copy(ei_hbm.at[pl.ds(0, 2), pl.ds(ebase, C)],
                         idx0, semI0).wait()
        pltpu.async_copy(ei_hbm.at[pl.ds(0, 2), pl.ds(ebase + C, C)],
                         idx1, semI1)
        pltpu.async_copy(kv_hbm.at[idx0.at[0]], kvb0, semG0)
        pltpu.async_copy(q_hbm.at[idx0.at[1]], qb0, semQ0)

        @pl.loop(0, CPT, step=2)
        def _(i):
            for b in range(2):
                ii = i + b
                idxb, kvbb, qbb, sdb, semIb, semGb, semQb, semDb = bufs[b]
                idxn, kvbn, qbn, sdn, semIn, semGn, semQn, semDn = bufs[1 - b]

                pltpu.async_copy(
                    ei_hbm.at[pl.ds(1, 1), pl.ds(ebase + ii * C, C)],
                    sdb, semDb)

                pltpu.make_async_copy(kv_hbm.at[idxb.at[0]], kvbb, semGb).wait()
                pltpu.make_async_copy(q_hbm.at[idxb.at[1]], qbb, semQb).wait()

                @pl.when(ii + 2 < CPT)
                def _():
                    pltpu.async_copy(
                        ei_hbm.at[pl.ds(0, 2), pl.ds(ebase + (ii + 2) * C, C)],
                        idxb, semIb)

                @pl.when(ii + 1 < CPT)
                def _():
                    pltpu.make_async_copy(
                        ei_hbm.at[pl.ds(0, 2), pl.ds(ebase, C)],
                        idxn, semIn).wait()
                    pltpu.async_copy(kv_hbm.at[idxn.at[0]], kvbn, semGn)
                    pltpu.async_copy(q_hbm.at[idxn.at[1]], qbn, semQn)

                @pl.when(ii > 0)
                def _():
                    pltpu.make_async_copy(mb, acc.at[sdn.at[0]], semS).wait()

                @plsc.parallel_loop(0, C, unroll=2)
                def _(e):
                    den = jnp.zeros((DH, ), jnp.float32)
                    for p in range(H // 2):
                        kpair = plsc.bitcast(kvbb[e, pl.ds(DH * p, DH)],
                                             jnp.bfloat16)
                        qpair = plsc.bitcast(qbb[e, pl.ds(DH * p, DH)],
                                             jnp.bfloat16)
                        kk = plsc.unpack(kpair, format=plsc.PackFormat.INTERLEAVED)
                        qq = plsc.unpack(qpair, format=plsc.PackFormat.INTERLEAVED)
                        for s in range(2):
                            h = 2 * p + s
                            evec = jnp.exp(jnp.broadcast_to(
                                jnp.sum(kk[s] * qq[s]), (DH,)))
                            vvec = kvbb[e, pl.ds(D // 2 + h * DH, DH)]
                            mb[e, pl.ds(h * DH, DH)] = vvec * evec
                            den = jnp.where(lane == h, evec, den)
                    mb[e, pl.ds(D, DH)] = den

                pltpu.make_async_copy(
                    ei_hbm.at[pl.ds(1, 1), pl.ds(ebase, C)], sdb, semDb).wait()
                pltpu.async_copy(mb, acc.at[sdb.at[0]], semS, add=True)

        pltpu.make_async_copy(mb, acc.at[sd1.at[0]], semS).wait()
        plsc.subcore_barrier()

        @pl.loop(0, ROWS_PER_TILE, step=OUT_STEP)
        def _(r):
            pltpu.sync_copy(acc.at[pl.ds(row0 + r, OUT_STEP)],
                            out_hbm.at[cid, pl.ds(row0 + r, OUT_STEP)])

    return sc_kernel(q_tab, kv_tab, edge_index)


def _combine(parts, W_o, b_o):
    BR = 1000

    def body(p_ref, wo_ref, bo_ref, o_ref):
        full = p_ref[0] + p_ref[1]
        msg = full[:, :D]
        r_ = lax.broadcasted_iota(jnp.int32, (ACC_W, D), 0)
        c_ = lax.broadcasted_iota(jnp.int32, (ACC_W, D), 1)
        expand = jnp.where((r_ >= D) & (c_ // DH == r_ - D), 1.0, 0.0)
        dexp = lax.dot_general(full, expand, (((1,), (0,)), ((), ())),
                               preferred_element_type=jnp.float32)
        attn_msg = msg / (dexp + 1e-16)
        out = lax.dot_general(attn_msg, wo_ref[...], (((1,), (1,)), ((), ())),
                              preferred_element_type=jnp.float32)
        o_ref[...] = out + bo_ref[...]

    return pl.pallas_call(
        body,
        grid=(N // BR,),
        in_specs=[pl.BlockSpec((NC, BR, ACC_W), lambda i: (0, i, 0)),
                  pl.BlockSpec((D, D), lambda i: (0, 0)),
                  pl.BlockSpec((1, D), lambda i: (0, 0))],
        out_specs=pl.BlockSpec((BR, D), lambda i: (i, 0)),
        out_shape=jax.ShapeDtypeStruct((N, D), jnp.float32),
    )(parts, W_o, b_o.reshape(1, D))


def _pack_pairs(x):
    n = x.shape[0]
    pairs = x.reshape(n, 4, 2, DH).transpose(0, 1, 3, 2).reshape(n, 64, 2)
    return jax.lax.bitcast_convert_type(pairs.astype(jnp.bfloat16), jnp.float32)


def kernel(q, kv, edge_index, W_k, b_k, W_v, b_v, W_o, b_o):
    kv_tab = _project_kv(kv, W_k, b_k, W_v, b_v)
    kv_mix = jnp.concatenate([_pack_pairs(kv_tab[:, :D]), kv_tab[:, D:]], axis=1)
    parts = _sc_attention(_pack_pairs(q), kv_mix, edge_index)
    return _combine(parts, W_o, b_o)
